# Initial kernel scaffold; baseline (speedup 1.0000x reference)
import jax
import jax.numpy as jnp
from jax.experimental import pallas as pl


def kernel(t, tab, proj_w):
    raise NotImplementedError("write your pallas kernel here")


# speedup vs baseline: 1.7991x; 1.7991x over previous
import functools

import jax
import jax.numpy as jnp
from jax import lax
from jax.experimental import pallas as pl
from jax.experimental.pallas import tpu as pltpu
from jax.experimental.pallas import tpu_sc as plsc

_SZ = 3072
_D = 1024
_MUL_T = 31337 % _SZ
_MUL_P = 1000003 % _SZ


def _proj_body(tab_ref, w_ref, out_ref):
    out_ref[...] = lax.dot_general(
        tab_ref[...], w_ref[...],
        dimension_numbers=(((1,), (1,)), ((), ())),
        preferred_element_type=jnp.float32)


def _project_table(tab, proj_w):
    m = tab.shape[0]
    bm = 512
    return pl.pallas_call(
        _proj_body,
        grid=(m // bm,),
        in_specs=[
            pl.BlockSpec((bm, _D), lambda i: (i, 0)),
            pl.BlockSpec((_D, _D), lambda i: (0, 0)),
        ],
        out_specs=pl.BlockSpec((bm, _D), lambda i: (i, 0)),
        out_shape=jax.ShapeDtypeStruct((m, _D), jnp.float32),
    )(tab, proj_w)


@functools.lru_cache(maxsize=None)
def _make_sc_gather(n):
    info = plsc.get_sparse_core_info()
    nc, ns = info.num_cores, info.num_subcores
    nw = nc * ns
    cb = n // nw
    k = 32
    nch = cb // k
    nvec = cb // 16
    mesh = plsc.VectorSubcoreMesh(core_axis_name="c", subcore_axis_name="s")

    @functools.partial(
        pl.kernel,
        mesh=mesh,
        out_type=jax.ShapeDtypeStruct((n, _D), jnp.float32),
        scratch_types=[
            pltpu.VMEM((cb,), jnp.int32),
            pltpu.VMEM((cb,), jnp.int32),
            pltpu.VMEM((cb,), jnp.int32),
            pltpu.VMEM((2, k, _D), jnp.float32),
            pltpu.SemaphoreType.DMA,
            pltpu.SemaphoreType.DMA,
            pltpu.SemaphoreType.DMA,
            pltpu.SemaphoreType.DMA,
        ],
    )
    def sc_gather(t_hbm, prev_hbm, ptab_hbm, out_hbm,
                  t_v, p_v, idx_v, rows, g0, g1, w0, w1):
        gsem = (g0, g1)
        wsem = (w0, w1)
        wid = lax.axis_index("s") * nc + lax.axis_index("c")
        base = wid * cb
        pltpu.sync_copy(t_hbm.at[pl.ds(base, cb)], t_v)
        pltpu.sync_copy(prev_hbm.at[pl.ds(base, cb)], p_v)

        def hash_body(i, carry):
            tv = t_v[pl.ds(i * 16, 16)]
            pv = p_v[pl.ds(i * 16, 16)]
            idx_v[pl.ds(i * 16, 16)] = (
                (tv % _SZ) * _MUL_T + (pv % _SZ) * _MUL_P) % _SZ
            return carry

        lax.fori_loop(0, nvec, hash_body, 0)

        def fire_gather(c, b):
            pltpu.async_copy(
                ptab_hbm.at[idx_v.at[pl.ds(c * k, k)]], rows.at[b], gsem[b])

        def wait_gather(c, b):
            pltpu.make_async_copy(
                ptab_hbm.at[idx_v.at[pl.ds(c * k, k)]], rows.at[b],
                gsem[b]).wait()

        def fire_write(c, b):
            pltpu.async_copy(
                rows.at[b], out_hbm.at[pl.ds(base + c * k, k)], wsem[b])

        def wait_write(c, b):
            pltpu.make_async_copy(
                rows.at[b], out_hbm.at[pl.ds(base + c * k, k)],
                wsem[b]).wait()

        for b in range(2):
            fire_gather(b, b)

        def round_body(i, carry):
            cbase = i * 2
            for b in range(2):
                c = cbase + b
                wait_gather(c, b)
                fire_write(c, b)
                wait_write(c, b)
                fire_gather(c + 2, b)
            return carry

        lax.fori_loop(0, (nch - 2) // 2, round_body, 0)

        for b in range(2):
            c = nch - 2 + b
            wait_gather(c, b)
            fire_write(c, b)
            wait_write(c, b)

    return sc_gather


def kernel(t, tab, proj_w):
    b, s = t.shape
    n = b * s
    ptab = _project_table(tab, proj_w)
    prev = jnp.pad(t[:, :-1], ((0, 0), (1, 0)))
    out = _make_sc_gather(n)(t.reshape(n), prev.reshape(n), ptab)
    return out.reshape(b, s, _D)
